# Initial kernel scaffold; baseline (speedup 1.0000x reference)
import jax
import jax.numpy as jnp
from jax.experimental import pallas as pl


def kernel(x, edge_index, W, b):
    raise NotImplementedError("write your pallas kernel here")


# speedup vs baseline: 9.6708x; 9.6708x over previous
import functools

import jax
import jax.numpy as jnp
from jax import lax
from jax.experimental import pallas as pl
from jax.experimental.pallas import tpu as pltpu
from jax.experimental.pallas import tpu_sc as plsc

N = 10000
E = 160000
D = 256
H = 128
NC, NS = 2, 16
NW = NC * NS
B = 128
EP = 163840
ROWS_ALL = EP // B
ROWS_A = ROWS_ALL // NW
ROWS_C = ROWS_ALL // NS
HALF = ROWS_C // 2
NBUF = 2
ACC_R = 10240
ZCH = ACC_R // NS
RB = 1000
NBANK = 8
HWORDS = NBANK * ACC_R

_MESH = plsc.VectorSubcoreMesh(core_axis_name="c", subcore_axis_name="s")
_SC_PARAMS = pltpu.CompilerParams(needs_layout_passes=False)


def _m8(v):
    return pl.multiple_of(v, 8)


@functools.partial(
    pl.kernel,
    out_type=jax.ShapeDtypeStruct((NW * HWORDS,), jnp.float32),
    mesh=_MESH,
    scratch_types=[
        pltpu.VMEM((HWORDS,), jnp.float32),
        pltpu.VMEM((ROWS_A * B,), jnp.int32),
    ],
    compiler_params=_SC_PARAMS,
)
def _deg_kernel(dstf_h, zflat_h, deg2_h, histf, dstg):
    c = lax.axis_index("c")
    s = lax.axis_index("s")
    w = s * NC + c
    pltpu.sync_copy(zflat_h, histf)
    pltpu.sync_copy(dstf_h.at[pl.ds(_m8(w * (ROWS_A * B)), ROWS_A * B)], dstg)

    lane = jnp.arange(16, dtype=jnp.int32)
    bank_base = (lane & 7) * ACC_R
    mask_a = lane < 8
    mask_b = lane >= 8
    ones16 = jnp.ones((16,), jnp.float32)

    def count_row(r, _):
        for j in range(B // 16):
            d = dstg[pl.ds(r * B + j * 16, 16)]
            addr = bank_base + d
            plsc.addupdate_scatter(histf, [addr], ones16, mask=mask_a)
            plsc.addupdate_scatter(histf, [addr], ones16, mask=mask_b)
        return 0

    lax.fori_loop(0, ROWS_A, count_row, 0)
    pltpu.sync_copy(histf, deg2_h.at[pl.ds(_m8(w * HWORDS), HWORDS)])


def _dense_body(x_ref, w_ref, b_ref, hist_ref, ones_ref,
                g_ref, sb_ref, dis_ref, deg_scr):
    i = pl.program_id(0)

    @pl.when(i == 0)
    def _():
        deg_scr[...] = lax.dot_general(
            hist_ref[...], ones_ref[...], (((0,), (0,)), ((), ())),
            precision=lax.Precision.HIGHEST,
            preferred_element_type=jnp.float32) + 1.0

    h = jnp.dot(x_ref[...], w_ref[...], preferred_element_type=jnp.float32)
    deg = deg_scr[pl.ds(_m8(i * RB), RB), :]
    dis = lax.rsqrt(deg)
    g = h * dis
    sb = h * (1.0 / deg) + b_ref[...]
    g_ref[0] = g[:, :H]
    g_ref[1] = g[:, H:]
    sb_ref[...] = sb
    dis_ref[...] = dis


def _dense(x, W, b2, hist, ones_col):
    return pl.pallas_call(
        _dense_body,
        grid=(N // RB,),
        in_specs=[
            pl.BlockSpec((RB, D), lambda i: (i, 0)),
            pl.BlockSpec((D, D), lambda i: (0, 0)),
            pl.BlockSpec((1, D), lambda i: (0, 0)),
            pl.BlockSpec((NW * NBANK, ACC_R), lambda i: (0, 0)),
            pl.BlockSpec((NW * NBANK, 1), lambda i: (0, 0)),
        ],
        scratch_shapes=[pltpu.VMEM((ACC_R, 1), jnp.float32)],
        out_specs=[
            pl.BlockSpec((NC, RB, H), lambda i: (0, i, 0)),
            pl.BlockSpec((RB, D), lambda i: (i, 0)),
            pl.BlockSpec((RB, 1), lambda i: (i, 0)),
        ],
        out_shape=[
            jax.ShapeDtypeStruct((NC, N, H), jnp.float32),
            jax.ShapeDtypeStruct((N, D), jnp.float32),
            jax.ShapeDtypeStruct((N, 1), jnp.float32),
        ],
    )(x, W, b2, hist, ones_col)


@functools.partial(
    pl.kernel,
    out_type=jax.ShapeDtypeStruct((ACC_R, D), jnp.float32),
    mesh=_MESH,
    scratch_types=[
        pltpu.VMEM_SHARED((ACC_R, H), jnp.float32),
        pltpu.VMEM((HALF, B), jnp.int32),
        pltpu.VMEM((HALF, B), jnp.int32),
        pltpu.VMEM((B, H), jnp.float32),
        pltpu.VMEM((B, H), jnp.float32),
        pltpu.SemaphoreType.DMA,
        pltpu.SemaphoreType.DMA,
        pltpu.SemaphoreType.DMA,
    ],
)
def _msg_kernel(srcp2_h, dstp_h, g_h, z_h, accd_h,
                acc, srcs, dsts, r0, r1, gs0, gs1, ssem):
    c = lax.axis_index("c")
    s = lax.axis_index("s")
    rbufs = [r0, r1]
    gsems = [gs0, gs1]
    pltpu.sync_copy(z_h, acc.at[pl.ds(_m8(s * ZCH), ZCH)])
    plsc.subcore_barrier()

    def edge_body(io, _):
        base = io * NBUF
        for bb in range(NBUF):
            pltpu.async_copy(g_h.at[srcs.at[base + bb]], rbufs[bb], gsems[bb])
        for bb in range(NBUF):
            pltpu.make_async_copy(
                g_h.at[srcs.at[base + bb]], rbufs[bb], gsems[bb]).wait()
            pltpu.async_copy(rbufs[bb], acc.at[dsts.at[base + bb]],
                             ssem, add=True)
        for bb in range(NBUF):
            pltpu.make_async_copy(
                rbufs[bb], acc.at[dsts.at[base + bb]], ssem).wait()
        return 0

    for half in range(2):
        pltpu.sync_copy(
            srcp2_h.at[pl.ds(_m8(c * ROWS_ALL + s * ROWS_C + half * HALF),
                             HALF)], srcs)
        pltpu.sync_copy(
            dstp_h.at[pl.ds(_m8(s * ROWS_C + half * HALF), HALF)], dsts)
        lax.fori_loop(0, HALF // NBUF, edge_body, 0)
    plsc.subcore_barrier()
    pltpu.sync_copy(
        acc.at[pl.ds(_m8(s * ZCH), ZCH)],
        accd_h.at[pl.ds(_m8(s * ZCH), ZCH),
                  pl.ds(pl.multiple_of(c * H, H), H)])


def _finish_body(a_ref, d_ref, s_ref, o_ref):
    o_ref[...] = a_ref[...] * d_ref[...] + s_ref[...]


def _finish(accd, discol, sb):
    return pl.pallas_call(
        _finish_body,
        grid=(N // RB,),
        in_specs=[
            pl.BlockSpec((RB, D), lambda i: (i, 0)),
            pl.BlockSpec((RB, 1), lambda i: (i, 0)),
            pl.BlockSpec((RB, D), lambda i: (i, 0)),
        ],
        out_specs=pl.BlockSpec((RB, D), lambda i: (i, 0)),
        out_shape=jax.ShapeDtypeStruct((N, D), jnp.float32),
    )(accd, discol, sb)


def kernel(x, edge_index, W, b):
    ei = edge_index.astype(jnp.int32)
    pad_src = jnp.zeros((EP - E,), jnp.int32)
    pad_dst = jnp.full((EP - E,), N, jnp.int32)
    src = jnp.concatenate([ei[0], pad_src])
    dstf = jnp.concatenate([ei[1], pad_dst])
    srcp2 = jnp.concatenate([src, src + N]).reshape(2 * ROWS_ALL, B)
    dstp = dstf.reshape(ROWS_ALL, B)
    zflat = jnp.zeros((HWORDS,), jnp.float32)
    zH = jnp.zeros((ZCH, H), jnp.float32)
    ones_col = jnp.ones((NW * NBANK, 1), jnp.float32)

    deg2 = _deg_kernel(dstf, zflat)
    hist = deg2.reshape(NW * NBANK, ACC_R)
    g3, sb, discol = _dense(x, W, b.reshape(1, D), hist, ones_col)
    g = g3.reshape(NC * N, H)
    accd = _msg_kernel(srcp2, dstp, g, zH)
    return _finish(accd, discol, sb)


# speedup vs baseline: 10.1345x; 1.0479x over previous
import functools

import jax
import jax.numpy as jnp
from jax import lax
from jax.experimental import pallas as pl
from jax.experimental.pallas import tpu as pltpu
from jax.experimental.pallas import tpu_sc as plsc

N = 10000
E = 160000
D = 256
H = 128
NC, NS = 2, 16
NW = NC * NS
B = 128
EP = 163840
ROWS_ALL = EP // B
ROWS_A = ROWS_ALL // NW
ROWS_C = ROWS_ALL // NS
HALF = ROWS_C // 2
NBUF = 2
ACC_R = 10240
ZCH = ACC_R // NS
RB = 1000
NBANK = 4
HWORDS = NBANK * ACC_R

_MESH = plsc.VectorSubcoreMesh(core_axis_name="c", subcore_axis_name="s")
_SC_PARAMS = pltpu.CompilerParams(needs_layout_passes=False)


def _m8(v):
    return pl.multiple_of(v, 8)


@functools.partial(
    pl.kernel,
    out_type=jax.ShapeDtypeStruct((NW * HWORDS,), jnp.float32),
    mesh=_MESH,
    scratch_types=[
        pltpu.VMEM((HWORDS,), jnp.float32),
        pltpu.VMEM((ROWS_A * B,), jnp.int32),
    ],
    compiler_params=_SC_PARAMS,
)
def _deg_kernel(dstf_h, zflat_h, deg2_h, histf, dstg):
    c = lax.axis_index("c")
    s = lax.axis_index("s")
    w = s * NC + c
    pltpu.sync_copy(zflat_h, histf)
    pltpu.sync_copy(dstf_h.at[pl.ds(_m8(w * (ROWS_A * B)), ROWS_A * B)], dstg)

    lane = jnp.arange(16, dtype=jnp.int32)
    bank_base = (lane & 3) * ACC_R
    group = lane >> 2
    masks = [group == k for k in range(4)]
    ones16 = jnp.ones((16,), jnp.float32)

    def count_row(r, _):
        for j in range(B // 16):
            d = dstg[pl.ds(r * B + j * 16, 16)]
            addr = bank_base + d
            for m in masks:
                plsc.addupdate_scatter(histf, [addr], ones16, mask=m)
        return 0

    lax.fori_loop(0, ROWS_A, count_row, 0)
    pltpu.sync_copy(histf, deg2_h.at[pl.ds(_m8(w * HWORDS), HWORDS)])


def _dense_body(x_ref, w_ref, b_ref, hist_ref, ones_ref,
                g_ref, sb_ref, dis_ref, deg_scr):
    i = pl.program_id(0)

    @pl.when(i == 0)
    def _():
        deg_scr[...] = lax.dot_general(
            hist_ref[...], ones_ref[...], (((0,), (0,)), ((), ())),
            precision=lax.Precision.HIGHEST,
            preferred_element_type=jnp.float32) + 1.0

    h = jnp.dot(x_ref[...], w_ref[...], preferred_element_type=jnp.float32)
    deg = deg_scr[pl.ds(_m8(i * RB), RB), :]
    dis = lax.rsqrt(deg)
    g = h * dis
    sb = h * (1.0 / deg) + b_ref[...]
    g_ref[0] = g[:, :H]
    g_ref[1] = g[:, H:]
    sb_ref[...] = sb
    dis_ref[...] = dis


def _dense(x, W, b2, hist, ones_col):
    return pl.pallas_call(
        _dense_body,
        grid=(N // RB,),
        in_specs=[
            pl.BlockSpec((RB, D), lambda i: (i, 0)),
            pl.BlockSpec((D, D), lambda i: (0, 0)),
            pl.BlockSpec((1, D), lambda i: (0, 0)),
            pl.BlockSpec((NW * NBANK, ACC_R), lambda i: (0, 0)),
            pl.BlockSpec((NW * NBANK, 1), lambda i: (0, 0)),
        ],
        scratch_shapes=[pltpu.VMEM((ACC_R, 1), jnp.float32)],
        out_specs=[
            pl.BlockSpec((NC, RB, H), lambda i: (0, i, 0)),
            pl.BlockSpec((RB, D), lambda i: (i, 0)),
            pl.BlockSpec((RB, 1), lambda i: (i, 0)),
        ],
        out_shape=[
            jax.ShapeDtypeStruct((NC, N, H), jnp.float32),
            jax.ShapeDtypeStruct((N, D), jnp.float32),
            jax.ShapeDtypeStruct((N, 1), jnp.float32),
        ],
    )(x, W, b2, hist, ones_col)


@functools.partial(
    pl.kernel,
    out_type=jax.ShapeDtypeStruct((ACC_R, D), jnp.float32),
    mesh=_MESH,
    scratch_types=[
        pltpu.VMEM_SHARED((ACC_R, H), jnp.float32),
        pltpu.VMEM((HALF, B), jnp.int32),
        pltpu.VMEM((HALF, B), jnp.int32),
        pltpu.VMEM((B, H), jnp.float32),
        pltpu.VMEM((B, H), jnp.float32),
        pltpu.SemaphoreType.DMA,
        pltpu.SemaphoreType.DMA,
        pltpu.SemaphoreType.DMA,
        pltpu.SemaphoreType.DMA,
    ],
)
def _msg_kernel(srcp2_h, dstp_h, g_h, z_h, accd_h,
                acc, srcs, dsts, r0, r1, gs0, gs1, ss0, ss1):
    c = lax.axis_index("c")
    s = lax.axis_index("s")
    rbufs = [r0, r1]
    gsems = [gs0, gs1]
    ssems = [ss0, ss1]
    pltpu.sync_copy(z_h, acc.at[pl.ds(_m8(s * ZCH), ZCH)])
    plsc.subcore_barrier()

    def _gather(j, bb):
        pltpu.async_copy(g_h.at[srcs.at[j]], rbufs[bb], gsems[bb])

    def _wait_gather(j, bb):
        pltpu.make_async_copy(g_h.at[srcs.at[j]], rbufs[bb], gsems[bb]).wait()

    def _scatter(j, bb):
        pltpu.async_copy(rbufs[bb], acc.at[dsts.at[j]], ssems[bb], add=True)

    def _wait_scatter(j, bb):
        pltpu.make_async_copy(rbufs[bb], acc.at[dsts.at[j]], ssems[bb]).wait()

    def edge_body(io, _):
        base = io * 2
        for bb in range(2):
            _wait_gather(base + bb, bb)
            _scatter(base + bb, bb)
        for bb in range(2):
            _wait_scatter(base + bb, bb)
            _gather(base + 2 + bb, bb)
        return 0

    for half in range(2):
        pltpu.sync_copy(
            srcp2_h.at[pl.ds(_m8(c * ROWS_ALL + s * ROWS_C + half * HALF),
                             HALF)], srcs)
        pltpu.sync_copy(
            dstp_h.at[pl.ds(_m8(s * ROWS_C + half * HALF), HALF)], dsts)
        for bb in range(2):
            _gather(bb, bb)
        lax.fori_loop(0, HALF // 2 - 1, edge_body, 0)
        for bb in range(2):
            _wait_gather(HALF - 2 + bb, bb)
            _scatter(HALF - 2 + bb, bb)
        for bb in range(2):
            _wait_scatter(HALF - 2 + bb, bb)
    plsc.subcore_barrier()
    pltpu.sync_copy(
        acc.at[pl.ds(_m8(s * ZCH), ZCH)],
        accd_h.at[pl.ds(_m8(s * ZCH), ZCH),
                  pl.ds(pl.multiple_of(c * H, H), H)])


def _finish_body(a_ref, d_ref, s_ref, o_ref):
    o_ref[...] = a_ref[...] * d_ref[...] + s_ref[...]


def _finish(accd, discol, sb):
    return pl.pallas_call(
        _finish_body,
        grid=(N // RB,),
        in_specs=[
            pl.BlockSpec((RB, D), lambda i: (i, 0)),
            pl.BlockSpec((RB, 1), lambda i: (i, 0)),
            pl.BlockSpec((RB, D), lambda i: (i, 0)),
        ],
        out_specs=pl.BlockSpec((RB, D), lambda i: (i, 0)),
        out_shape=jax.ShapeDtypeStruct((N, D), jnp.float32),
    )(accd, discol, sb)


def kernel(x, edge_index, W, b):
    ei = edge_index.astype(jnp.int32)
    pad_src = jnp.zeros((EP - E,), jnp.int32)
    pad_dst = jnp.full((EP - E,), N, jnp.int32)
    src = jnp.concatenate([ei[0], pad_src])
    dstf = jnp.concatenate([ei[1], pad_dst])
    srcp2 = jnp.concatenate([src, src + N]).reshape(2 * ROWS_ALL, B)
    dstp = dstf.reshape(ROWS_ALL, B)
    zflat = jnp.zeros((HWORDS,), jnp.float32)
    zH = jnp.zeros((ZCH, H), jnp.float32)
    ones_col = jnp.ones((NW * NBANK, 1), jnp.float32)

    deg2 = _deg_kernel(dstf, zflat)
    hist = deg2.reshape(NW * NBANK, ACC_R)
    g3, sb, discol = _dense(x, W, b.reshape(1, D), hist, ones_col)
    g = g3.reshape(NC * N, H)
    accd = _msg_kernel(srcp2, dstp, g, zH)
    return _finish(accd, discol, sb)


# speedup vs baseline: 10.5473x; 1.0407x over previous
import functools

import jax
import jax.numpy as jnp
from jax import lax
from jax.experimental import pallas as pl
from jax.experimental.pallas import tpu as pltpu
from jax.experimental.pallas import tpu_sc as plsc

N = 10000
E = 160000
D = 256
H = 128
NC, NS = 2, 16
NW = NC * NS
B = 80
EP = 163840
BQ = EP // B
RT = BQ // NS
NSTAGE = 4
SQ = RT // NSTAGE
NBUF = 4
BA = 128
ROWS_A = EP // BA // NW
ACC_R = 10112
ZCH = ACC_R // NS
RB = 1000
NBANK = 4
HWORDS = NBANK * ACC_R

_MESH = plsc.VectorSubcoreMesh(core_axis_name="c", subcore_axis_name="s")
_SC_PARAMS = pltpu.CompilerParams(needs_layout_passes=False)


def _m8(v):
    return pl.multiple_of(v, 8)


@functools.partial(
    pl.kernel,
    out_type=jax.ShapeDtypeStruct((NW * HWORDS,), jnp.float32),
    mesh=_MESH,
    scratch_types=[
        pltpu.VMEM((HWORDS,), jnp.float32),
        pltpu.VMEM((ROWS_A * BA,), jnp.int32),
    ],
    compiler_params=_SC_PARAMS,
)
def _deg_kernel(dstf_h, zflat_h, deg2_h, histf, dstg):
    c = lax.axis_index("c")
    s = lax.axis_index("s")
    w = s * NC + c
    pltpu.sync_copy(zflat_h, histf)
    pltpu.sync_copy(dstf_h.at[pl.ds(_m8(w * (ROWS_A * BA)), ROWS_A * BA)], dstg)

    lane = jnp.arange(16, dtype=jnp.int32)
    bank_base = (lane & 3) * ACC_R
    group = lane >> 2
    masks = [group == k for k in range(4)]
    ones16 = jnp.ones((16,), jnp.float32)

    def count_row(r, _):
        for j in range(BA // 16):
            d = dstg[pl.ds(r * BA + j * 16, 16)]
            addr = bank_base + d
            for m in masks:
                plsc.addupdate_scatter(histf, [addr], ones16, mask=m)
        return 0

    lax.fori_loop(0, ROWS_A, count_row, 0)
    pltpu.sync_copy(histf, deg2_h.at[pl.ds(_m8(w * HWORDS), HWORDS)])


def _dense_body(x_ref, w_ref, b_ref, hist_ref, ones_ref,
                g_ref, sb_ref, dis_ref, deg_scr):
    i = pl.program_id(0)

    @pl.when(i == 0)
    def _():
        deg_scr[...] = lax.dot_general(
            hist_ref[...], ones_ref[...], (((0,), (0,)), ((), ())),
            precision=lax.Precision.HIGHEST,
            preferred_element_type=jnp.float32) + 1.0

    h = jnp.dot(x_ref[...], w_ref[...], preferred_element_type=jnp.float32)
    deg = deg_scr[pl.ds(_m8(i * RB), RB), :]
    dis = lax.rsqrt(deg)
    g = h * dis
    sb = h * (1.0 / deg) + b_ref[...]
    g_ref[0] = g[:, :H]
    g_ref[1] = g[:, H:]
    sb_ref[...] = sb
    dis_ref[...] = dis


def _dense(x, W, b2, hist, ones_col):
    return pl.pallas_call(
        _dense_body,
        grid=(N // RB,),
        in_specs=[
            pl.BlockSpec((RB, D), lambda i: (i, 0)),
            pl.BlockSpec((D, D), lambda i: (0, 0)),
            pl.BlockSpec((1, D), lambda i: (0, 0)),
            pl.BlockSpec((NW * NBANK, ACC_R), lambda i: (0, 0)),
            pl.BlockSpec((NW * NBANK, 1), lambda i: (0, 0)),
        ],
        scratch_shapes=[pltpu.VMEM((ACC_R, 1), jnp.float32)],
        out_specs=[
            pl.BlockSpec((NC, RB, H), lambda i: (0, i, 0)),
            pl.BlockSpec((RB, D), lambda i: (i, 0)),
            pl.BlockSpec((RB, 1), lambda i: (i, 0)),
        ],
        out_shape=[
            jax.ShapeDtypeStruct((NC, N, H), jnp.float32),
            jax.ShapeDtypeStruct((N, D), jnp.float32),
            jax.ShapeDtypeStruct((N, 1), jnp.float32),
        ],
    )(x, W, b2, hist, ones_col)


@functools.partial(
    pl.kernel,
    out_type=jax.ShapeDtypeStruct((ACC_R, D), jnp.float32),
    mesh=_MESH,
    scratch_types=[
        pltpu.VMEM_SHARED((ACC_R, H), jnp.float32),
        pltpu.VMEM((SQ, B), jnp.int32),
        pltpu.VMEM((SQ, B), jnp.int32),
        pltpu.VMEM((B, H), jnp.float32),
        pltpu.VMEM((B, H), jnp.float32),
        pltpu.VMEM((B, H), jnp.float32),
        pltpu.VMEM((B, H), jnp.float32),
        pltpu.SemaphoreType.DMA,
        pltpu.SemaphoreType.DMA,
        pltpu.SemaphoreType.DMA,
        pltpu.SemaphoreType.DMA,
        pltpu.SemaphoreType.DMA,
        pltpu.SemaphoreType.DMA,
        pltpu.SemaphoreType.DMA,
        pltpu.SemaphoreType.DMA,
    ],
)
def _msg_kernel(srcp2_h, dstp_h, g_h, z_h, accd_h,
                acc, srcs, dsts, r0, r1, r2, r3,
                gs0, gs1, gs2, gs3, ss0, ss1, ss2, ss3):
    c = lax.axis_index("c")
    s = lax.axis_index("s")
    rbufs = [r0, r1, r2, r3]
    gsems = [gs0, gs1, gs2, gs3]
    ssems = [ss0, ss1, ss2, ss3]
    pltpu.sync_copy(z_h, acc.at[pl.ds(_m8(s * ZCH), ZCH)])
    plsc.subcore_barrier()

    def _gather(j, bb):
        pltpu.async_copy(g_h.at[srcs.at[j]], rbufs[bb], gsems[bb])

    def _wait_gather(j, bb):
        pltpu.make_async_copy(g_h.at[srcs.at[j]], rbufs[bb], gsems[bb]).wait()

    def _scatter(j, bb):
        pltpu.async_copy(rbufs[bb], acc.at[dsts.at[j]], ssems[bb], add=True)

    def _wait_scatter(j, bb):
        pltpu.make_async_copy(rbufs[bb], acc.at[dsts.at[j]], ssems[bb]).wait()

    def edge_body(io, _):
        base = io * NBUF
        for bb in range(NBUF):
            _wait_gather(base + bb, bb)
            _scatter(base + bb, bb)
        for bb in range(NBUF):
            _wait_scatter(base + bb, bb)
            _gather(base + NBUF + bb, bb)
        return 0

    for q in range(NSTAGE):
        pltpu.sync_copy(
            srcp2_h.at[pl.ds(_m8(c * BQ + s * RT + q * SQ), SQ)], srcs)
        pltpu.sync_copy(
            dstp_h.at[pl.ds(_m8(s * RT + q * SQ), SQ)], dsts)
        for bb in range(NBUF):
            _gather(bb, bb)
        lax.fori_loop(0, SQ // NBUF - 1, edge_body, 0)
        for bb in range(NBUF):
            _wait_gather(SQ - NBUF + bb, bb)
            _scatter(SQ - NBUF + bb, bb)
        for bb in range(NBUF):
            _wait_scatter(SQ - NBUF + bb, bb)
    plsc.subcore_barrier()
    pltpu.sync_copy(
        acc.at[pl.ds(_m8(s * ZCH), ZCH)],
        accd_h.at[pl.ds(_m8(s * ZCH), ZCH),
                  pl.ds(pl.multiple_of(c * H, H), H)])


def _finish_body(a_ref, d_ref, s_ref, o_ref):
    o_ref[...] = a_ref[...] * d_ref[...] + s_ref[...]


def _finish(accd, discol, sb):
    return pl.pallas_call(
        _finish_body,
        grid=(N // RB,),
        in_specs=[
            pl.BlockSpec((RB, D), lambda i: (i, 0)),
            pl.BlockSpec((RB, 1), lambda i: (i, 0)),
            pl.BlockSpec((RB, D), lambda i: (i, 0)),
        ],
        out_specs=pl.BlockSpec((RB, D), lambda i: (i, 0)),
        out_shape=jax.ShapeDtypeStruct((N, D), jnp.float32),
    )(accd, discol, sb)


def kernel(x, edge_index, W, b):
    ei = edge_index.astype(jnp.int32)
    pad_src = jnp.zeros((EP - E,), jnp.int32)
    pad_dst = jnp.full((EP - E,), N, jnp.int32)
    src = jnp.concatenate([ei[0], pad_src])
    dstf = jnp.concatenate([ei[1], pad_dst])
    srcp2 = jnp.concatenate([src, src + N]).reshape(2 * BQ, B)
    dstp = dstf.reshape(BQ, B)
    zflat = jnp.zeros((HWORDS,), jnp.float32)
    zH = jnp.zeros((ZCH, H), jnp.float32)
    ones_col = jnp.ones((NW * NBANK, 1), jnp.float32)

    deg2 = _deg_kernel(dstf, zflat)
    hist = deg2.reshape(NW * NBANK, ACC_R)
    g3, sb, discol = _dense(x, W, b.reshape(1, D), hist, ones_col)
    g = g3.reshape(NC * N, H)
    accd = _msg_kernel(srcp2, dstp, g, zH)
    return _finish(accd, discol, sb)


# speedup vs baseline: 11.8504x; 1.1235x over previous
import functools

import jax
import jax.numpy as jnp
from jax import lax
from jax.experimental import pallas as pl
from jax.experimental.pallas import tpu as pltpu
from jax.experimental.pallas import tpu_sc as plsc

N = 10000
E = 160000
D = 256
H = 128
NC, NS = 2, 16
NW = NC * NS
B = 80
EP = 163840
BQ = EP // B
RT = BQ // NS
NSTAGE = 4
SQ = RT // NSTAGE
NBUF = 4
BA = 128
ROWS_A = EP // BA // NW
ACC_R = 10112
ZCH = ACC_R // NS
RB = 1000
NBANK = 4
HWORDS = NBANK * ACC_R

_MESH = plsc.VectorSubcoreMesh(core_axis_name="c", subcore_axis_name="s")
_SC_PARAMS = pltpu.CompilerParams(needs_layout_passes=False)


def _m8(v):
    return pl.multiple_of(v, 8)


@functools.partial(
    pl.kernel,
    out_type=jax.ShapeDtypeStruct((NW * HWORDS,), jnp.float32),
    mesh=_MESH,
    scratch_types=[
        pltpu.VMEM((HWORDS,), jnp.float32),
        pltpu.VMEM((ROWS_A * BA,), jnp.int32),
    ],
    compiler_params=_SC_PARAMS,
)
def _deg_kernel(dstf_h, zflat_h, deg2_h, histf, dstg):
    c = lax.axis_index("c")
    s = lax.axis_index("s")
    w = s * NC + c
    pltpu.sync_copy(zflat_h, histf)
    pltpu.sync_copy(dstf_h.at[pl.ds(_m8(w * (ROWS_A * BA)), ROWS_A * BA)], dstg)

    lane = jnp.arange(16, dtype=jnp.int32)
    bank_base = (lane & 3) * ACC_R
    group = lane >> 2
    masks = [group == k for k in range(4)]
    ones16 = jnp.ones((16,), jnp.float32)

    def count_row(r, _):
        for j in range(BA // 16):
            d = dstg[pl.ds(r * BA + j * 16, 16)]
            addr = bank_base + d
            for m in masks:
                plsc.addupdate_scatter(histf, [addr], ones16, mask=m)
        return 0

    lax.fori_loop(0, ROWS_A, count_row, 0)
    pltpu.sync_copy(histf, deg2_h.at[pl.ds(_m8(w * HWORDS), HWORDS)])


def _matmul_body(x_ref, w_ref, h_ref):
    h_ref[...] = jnp.dot(x_ref[...], w_ref[...],
                         preferred_element_type=jnp.float32)


def _matmul(x, W):
    return pl.pallas_call(
        _matmul_body,
        grid=(N // RB,),
        in_specs=[
            pl.BlockSpec((RB, D), lambda i: (i, 0)),
            pl.BlockSpec((D, D), lambda i: (0, 0)),
        ],
        out_specs=pl.BlockSpec((RB, D), lambda i: (i, 0)),
        out_shape=jax.ShapeDtypeStruct((N, D), jnp.float32),
    )(x, W)


def _post_body(h_ref, hist_ref, ones_ref, g_ref, dis_ref, deg_scr):
    i = pl.program_id(0)

    @pl.when(i == 0)
    def _():
        deg_scr[...] = lax.dot_general(
            hist_ref[...], ones_ref[...], (((0,), (0,)), ((), ())),
            precision=lax.Precision.HIGHEST,
            preferred_element_type=jnp.float32) + 1.0

    dis = lax.rsqrt(deg_scr[pl.ds(_m8(i * RB), RB), :])
    g = h_ref[...] * dis
    g_ref[0] = g[:, :H]
    g_ref[1] = g[:, H:]
    dis_ref[...] = dis


def _post(h, hist, ones_col):
    return pl.pallas_call(
        _post_body,
        grid=(N // RB,),
        in_specs=[
            pl.BlockSpec((RB, D), lambda i: (i, 0)),
            pl.BlockSpec((NW * NBANK, ACC_R), lambda i: (0, 0)),
            pl.BlockSpec((NW * NBANK, 1), lambda i: (0, 0)),
        ],
        scratch_shapes=[pltpu.VMEM((ACC_R, 1), jnp.float32)],
        out_specs=[
            pl.BlockSpec((NC, RB, H), lambda i: (0, i, 0)),
            pl.BlockSpec((RB, 1), lambda i: (i, 0)),
        ],
        out_shape=[
            jax.ShapeDtypeStruct((NC, N, H), jnp.float32),
            jax.ShapeDtypeStruct((N, 1), jnp.float32),
        ],
    )(h, hist, ones_col)


@functools.partial(
    pl.kernel,
    out_type=jax.ShapeDtypeStruct((ACC_R, D), jnp.float32),
    mesh=_MESH,
    scratch_types=[
        pltpu.VMEM_SHARED((ACC_R, H), jnp.float32),
        pltpu.VMEM((SQ, B), jnp.int32),
        pltpu.VMEM((SQ, B), jnp.int32),
        pltpu.VMEM((B, H), jnp.float32),
        pltpu.VMEM((B, H), jnp.float32),
        pltpu.VMEM((B, H), jnp.float32),
        pltpu.VMEM((B, H), jnp.float32),
        pltpu.SemaphoreType.DMA,
        pltpu.SemaphoreType.DMA,
        pltpu.SemaphoreType.DMA,
        pltpu.SemaphoreType.DMA,
        pltpu.SemaphoreType.DMA,
        pltpu.SemaphoreType.DMA,
        pltpu.SemaphoreType.DMA,
        pltpu.SemaphoreType.DMA,
    ],
)
def _msg_kernel(srcp2_h, dstp_h, g_h, z_h, accd_h,
                acc, srcs, dsts, r0, r1, r2, r3,
                gs0, gs1, gs2, gs3, ss0, ss1, ss2, ss3):
    c = lax.axis_index("c")
    s = lax.axis_index("s")
    rbufs = [r0, r1, r2, r3]
    gsems = [gs0, gs1, gs2, gs3]
    ssems = [ss0, ss1, ss2, ss3]
    pltpu.sync_copy(z_h, acc.at[pl.ds(_m8(s * ZCH), ZCH)])
    plsc.subcore_barrier()

    def _gather(j, bb):
        pltpu.async_copy(g_h.at[srcs.at[j]], rbufs[bb], gsems[bb])

    def _wait_gather(j, bb):
        pltpu.make_async_copy(g_h.at[srcs.at[j]], rbufs[bb], gsems[bb]).wait()

    def _scatter(j, bb):
        pltpu.async_copy(rbufs[bb], acc.at[dsts.at[j]], ssems[bb], add=True)

    def _wait_scatter(j, bb):
        pltpu.make_async_copy(rbufs[bb], acc.at[dsts.at[j]], ssems[bb]).wait()

    def edge_body(io, _):
        base = io * NBUF
        for bb in range(NBUF):
            _wait_gather(base + bb, bb)
            _scatter(base + bb, bb)
        for bb in range(NBUF):
            _wait_scatter(base + bb, bb)
            _gather(base + NBUF + bb, bb)
        return 0

    for q in range(NSTAGE):
        pltpu.sync_copy(
            srcp2_h.at[pl.ds(_m8(c * BQ + s * RT + q * SQ), SQ)], srcs)
        pltpu.sync_copy(
            dstp_h.at[pl.ds(_m8(s * RT + q * SQ), SQ)], dsts)
        for bb in range(NBUF):
            _gather(bb, bb)
        lax.fori_loop(0, SQ // NBUF - 1, edge_body, 0)
        for bb in range(NBUF):
            _wait_gather(SQ - NBUF + bb, bb)
            _scatter(SQ - NBUF + bb, bb)
        for bb in range(NBUF):
            _wait_scatter(SQ - NBUF + bb, bb)
    plsc.subcore_barrier()
    pltpu.sync_copy(
        acc.at[pl.ds(_m8(s * ZCH), ZCH)],
        accd_h.at[pl.ds(_m8(s * ZCH), ZCH),
                  pl.ds(pl.multiple_of(c * H, H), H)])


def _finish_body(a_ref, g0_ref, g1_ref, d_ref, b_ref, o_ref):
    g = jnp.concatenate([g0_ref[0], g1_ref[0]], axis=1)
    o_ref[...] = (a_ref[...] + g) * d_ref[...] + b_ref[...]


def _finish(accd, g3, discol, b2):
    return pl.pallas_call(
        _finish_body,
        grid=(N // RB,),
        in_specs=[
            pl.BlockSpec((RB, D), lambda i: (i, 0)),
            pl.BlockSpec((1, RB, H), lambda i: (0, i, 0)),
            pl.BlockSpec((1, RB, H), lambda i: (1, i, 0)),
            pl.BlockSpec((RB, 1), lambda i: (i, 0)),
            pl.BlockSpec((1, D), lambda i: (0, 0)),
        ],
        out_specs=pl.BlockSpec((RB, D), lambda i: (i, 0)),
        out_shape=jax.ShapeDtypeStruct((N, D), jnp.float32),
    )(accd, g3, g3, discol, b2)


def kernel(x, edge_index, W, b):
    ei = edge_index.astype(jnp.int32)
    pad_src = jnp.zeros((EP - E,), jnp.int32)
    pad_dst = jnp.full((EP - E,), N, jnp.int32)
    src = jnp.concatenate([ei[0], pad_src])
    dstf = jnp.concatenate([ei[1], pad_dst])
    srcp2 = jnp.concatenate([src, src + N]).reshape(2 * BQ, B)
    dstp = dstf.reshape(BQ, B)
    zflat = jnp.zeros((HWORDS,), jnp.float32)
    zH = jnp.zeros((ZCH, H), jnp.float32)
    ones_col = jnp.ones((NW * NBANK, 1), jnp.float32)

    deg2 = _deg_kernel(dstf, zflat)
    h = _matmul(x, W)
    hist = deg2.reshape(NW * NBANK, ACC_R)
    g3, discol = _post(h, hist, ones_col)
    g = g3.reshape(NC * N, H)
    accd = _msg_kernel(srcp2, dstp, g, zH)
    return _finish(accd, g3, discol, b.reshape(1, D))


# speedup vs baseline: 11.9092x; 1.0050x over previous
import functools

import jax
import jax.numpy as jnp
from jax import lax
from jax.experimental import pallas as pl
from jax.experimental.pallas import tpu as pltpu
from jax.experimental.pallas import tpu_sc as plsc

N = 10000
E = 160000
D = 256
H = 128
NC, NS = 2, 16
NW = NC * NS
B = 80
EP = 163840
BQ = EP // B
RT = BQ // NS
NSTAGE = 8
SQ = RT // NSTAGE
NBUF = 4
BA = 128
ROWS_A = EP // BA // NW
ACC_R = 10112
ZCH = ACC_R // NS
RB = 1000
NBANK = 4
HWORDS = NBANK * ACC_R

_MESH = plsc.VectorSubcoreMesh(core_axis_name="c", subcore_axis_name="s")
_SC_PARAMS = pltpu.CompilerParams(needs_layout_passes=False)


def _m8(v):
    return pl.multiple_of(v, 8)


@functools.partial(
    pl.kernel,
    out_type=jax.ShapeDtypeStruct((NW * HWORDS,), jnp.float32),
    mesh=_MESH,
    scratch_types=[
        pltpu.VMEM((HWORDS,), jnp.float32),
        pltpu.VMEM((ROWS_A * BA,), jnp.int32),
    ],
    compiler_params=_SC_PARAMS,
)
def _deg_kernel(dstf_h, zflat_h, deg2_h, histf, dstg):
    c = lax.axis_index("c")
    s = lax.axis_index("s")
    w = s * NC + c
    pltpu.sync_copy(zflat_h, histf)
    pltpu.sync_copy(dstf_h.at[pl.ds(_m8(w * (ROWS_A * BA)), ROWS_A * BA)], dstg)

    lane = jnp.arange(16, dtype=jnp.int32)
    bank_base = (lane & 3) * ACC_R
    group = lane >> 2
    masks = [group == k for k in range(4)]
    ones16 = jnp.ones((16,), jnp.float32)

    def count_row(r, _):
        for j in range(BA // 16):
            d = dstg[pl.ds(r * BA + j * 16, 16)]
            addr = bank_base + d
            for m in masks:
                plsc.addupdate_scatter(histf, [addr], ones16, mask=m)
        return 0

    lax.fori_loop(0, ROWS_A, count_row, 0)
    pltpu.sync_copy(histf, deg2_h.at[pl.ds(_m8(w * HWORDS), HWORDS)])


def _matmul_body(x_ref, w_ref, h_ref):
    h_ref[...] = jnp.dot(x_ref[...], w_ref[...],
                         preferred_element_type=jnp.float32)


def _matmul(x, W):
    return pl.pallas_call(
        _matmul_body,
        grid=(N // RB,),
        in_specs=[
            pl.BlockSpec((RB, D), lambda i: (i, 0)),
            pl.BlockSpec((D, D), lambda i: (0, 0)),
        ],
        out_specs=pl.BlockSpec((RB, D), lambda i: (i, 0)),
        out_shape=jax.ShapeDtypeStruct((N, D), jnp.float32),
    )(x, W)


def _post_body(h_ref, hist_ref, ones_ref, g_ref, dis_ref, deg_scr):
    i = pl.program_id(0)

    @pl.when(i == 0)
    def _():
        deg_scr[...] = lax.dot_general(
            hist_ref[...], ones_ref[...], (((0,), (0,)), ((), ())),
            precision=lax.Precision.HIGHEST,
            preferred_element_type=jnp.float32) + 1.0

    dis = lax.rsqrt(deg_scr[pl.ds(_m8(i * RB), RB), :])
    g = h_ref[...] * dis
    g_ref[0] = g[:, :H]
    g_ref[1] = g[:, H:]
    dis_ref[...] = dis


def _post(h, hist, ones_col):
    return pl.pallas_call(
        _post_body,
        grid=(N // RB,),
        in_specs=[
            pl.BlockSpec((RB, D), lambda i: (i, 0)),
            pl.BlockSpec((NW * NBANK, ACC_R), lambda i: (0, 0)),
            pl.BlockSpec((NW * NBANK, 1), lambda i: (0, 0)),
        ],
        scratch_shapes=[pltpu.VMEM((ACC_R, 1), jnp.float32)],
        out_specs=[
            pl.BlockSpec((NC, RB, H), lambda i: (0, i, 0)),
            pl.BlockSpec((RB, 1), lambda i: (i, 0)),
        ],
        out_shape=[
            jax.ShapeDtypeStruct((NC, N, H), jnp.float32),
            jax.ShapeDtypeStruct((N, 1), jnp.float32),
        ],
    )(h, hist, ones_col)


@functools.partial(
    pl.kernel,
    out_type=jax.ShapeDtypeStruct((ACC_R, D), jnp.float32),
    mesh=_MESH,
    scratch_types=[
        pltpu.VMEM_SHARED((ACC_R, H), jnp.float32),
        pltpu.VMEM((SQ, B), jnp.int32),
        pltpu.VMEM((SQ, B), jnp.int32),
        pltpu.VMEM((SQ, B), jnp.int32),
        pltpu.VMEM((SQ, B), jnp.int32),
        pltpu.VMEM((B, H), jnp.float32),
        pltpu.VMEM((B, H), jnp.float32),
        pltpu.VMEM((B, H), jnp.float32),
        pltpu.VMEM((B, H), jnp.float32),
        pltpu.SemaphoreType.DMA,
        pltpu.SemaphoreType.DMA,
        pltpu.SemaphoreType.DMA,
        pltpu.SemaphoreType.DMA,
        pltpu.SemaphoreType.DMA,
        pltpu.SemaphoreType.DMA,
        pltpu.SemaphoreType.DMA,
        pltpu.SemaphoreType.DMA,
        pltpu.SemaphoreType.DMA,
        pltpu.SemaphoreType.DMA,
    ],
)
def _msg_kernel(srcp2_h, dstp_h, g_h, z_h, accd_h,
                acc, srcsA, dstsA, srcsB, dstsB, r0, r1, r2, r3,
                gs0, gs1, gs2, gs3, ss0, ss1, ss2, ss3, is0, is1):
    c = lax.axis_index("c")
    s = lax.axis_index("s")
    rbufs = [r0, r1, r2, r3]
    gsems = [gs0, gs1, gs2, gs3]
    ssems = [ss0, ss1, ss2, ss3]
    ibufs = [(srcsA, dstsA), (srcsB, dstsB)]
    isems = [is0, is1]

    def _stage(q, pair, isem):
        sbuf, dbuf = ibufs[pair]
        pltpu.async_copy(
            srcp2_h.at[pl.ds(_m8(c * BQ + s * RT + q * SQ), SQ)], sbuf,
            isems[isem])
        pltpu.async_copy(
            dstp_h.at[pl.ds(_m8(s * RT + q * SQ), SQ)], dbuf, isems[isem])

    def _wait_stage(q, pair, isem):
        sbuf, dbuf = ibufs[pair]
        pltpu.make_async_copy(
            srcp2_h.at[pl.ds(_m8(c * BQ + s * RT + q * SQ), SQ)], sbuf,
            isems[isem]).wait()
        pltpu.make_async_copy(
            dstp_h.at[pl.ds(_m8(s * RT + q * SQ), SQ)], dbuf,
            isems[isem]).wait()

    _stage(0, 0, 0)
    pltpu.sync_copy(z_h, acc.at[pl.ds(_m8(s * ZCH), ZCH)])
    plsc.subcore_barrier()

    def _gather(srcs, j, bb):
        pltpu.async_copy(g_h.at[srcs.at[j]], rbufs[bb], gsems[bb])

    def _wait_gather(srcs, j, bb):
        pltpu.make_async_copy(g_h.at[srcs.at[j]], rbufs[bb], gsems[bb]).wait()

    def _scatter(dsts, j, bb):
        pltpu.async_copy(rbufs[bb], acc.at[dsts.at[j]], ssems[bb], add=True)

    def _wait_scatter(dsts, j, bb):
        pltpu.make_async_copy(rbufs[bb], acc.at[dsts.at[j]], ssems[bb]).wait()

    for q in range(NSTAGE):
        pair = q % 2
        srcs, dsts = ibufs[pair]
        _wait_stage(q, pair, pair)
        if q + 1 < NSTAGE:
            _stage(q + 1, 1 - pair, 1 - pair)

        def edge_body(io, _, srcs=srcs, dsts=dsts):
            base = io * NBUF
            for bb in range(NBUF):
                _wait_gather(srcs, base + bb, bb)
                _scatter(dsts, base + bb, bb)
            for bb in range(NBUF):
                _wait_scatter(dsts, base + bb, bb)
                _gather(srcs, base + NBUF + bb, bb)
            return 0

        for bb in range(NBUF):
            _gather(srcs, bb, bb)
        lax.fori_loop(0, SQ // NBUF - 1, edge_body, 0)
        for bb in range(NBUF):
            _wait_gather(srcs, SQ - NBUF + bb, bb)
            _scatter(dsts, SQ - NBUF + bb, bb)
        for bb in range(NBUF):
            _wait_scatter(dsts, SQ - NBUF + bb, bb)
    plsc.subcore_barrier()
    pltpu.sync_copy(
        acc.at[pl.ds(_m8(s * ZCH), ZCH)],
        accd_h.at[pl.ds(_m8(s * ZCH), ZCH),
                  pl.ds(pl.multiple_of(c * H, H), H)])


def _finish_body(a_ref, g0_ref, g1_ref, d_ref, b_ref, o_ref):
    g = jnp.concatenate([g0_ref[0], g1_ref[0]], axis=1)
    o_ref[...] = (a_ref[...] + g) * d_ref[...] + b_ref[...]


def _finish(accd, g3, discol, b2):
    return pl.pallas_call(
        _finish_body,
        grid=(N // RB,),
        in_specs=[
            pl.BlockSpec((RB, D), lambda i: (i, 0)),
            pl.BlockSpec((1, RB, H), lambda i: (0, i, 0)),
            pl.BlockSpec((1, RB, H), lambda i: (1, i, 0)),
            pl.BlockSpec((RB, 1), lambda i: (i, 0)),
            pl.BlockSpec((1, D), lambda i: (0, 0)),
        ],
        out_specs=pl.BlockSpec((RB, D), lambda i: (i, 0)),
        out_shape=jax.ShapeDtypeStruct((N, D), jnp.float32),
    )(accd, g3, g3, discol, b2)


def kernel(x, edge_index, W, b):
    ei = edge_index.astype(jnp.int32)
    pad_src = jnp.zeros((EP - E,), jnp.int32)
    pad_dst = jnp.full((EP - E,), N, jnp.int32)
    src = jnp.concatenate([ei[0], pad_src])
    dstf = jnp.concatenate([ei[1], pad_dst])
    srcp2 = jnp.concatenate([src, src + N]).reshape(2 * BQ, B)
    dstp = dstf.reshape(BQ, B)
    zflat = jnp.zeros((HWORDS,), jnp.float32)
    zH = jnp.zeros((ZCH, H), jnp.float32)
    ones_col = jnp.ones((NW * NBANK, 1), jnp.float32)

    deg2 = _deg_kernel(dstf, zflat)
    h = _matmul(x, W)
    hist = deg2.reshape(NW * NBANK, ACC_R)
    g3, discol = _post(h, hist, ones_col)
    g = g3.reshape(NC * N, H)
    accd = _msg_kernel(srcp2, dstp, g, zH)
    return _finish(accd, g3, discol, b.reshape(1, D))
